# Initial kernel scaffold; baseline (speedup 1.0000x reference)
import jax
import jax.numpy as jnp
from jax.experimental import pallas as pl


def kernel(hidden_states, position_ids, w_q_a, q_a_norm_w, w_q_b, w_kv_a, kv_a_norm_w, w_kv_b, w_o, w_idx_qb, w_idx_k, idx_k_norm_w, idx_k_norm_b, w_idx_w):
    raise NotImplementedError("write your pallas kernel here")


# speedup vs baseline: 3.6098x; 3.6098x over previous
import jax
import jax.numpy as jnp
from jax.experimental import pallas as pl
from jax.experimental.pallas import tpu as pltpu

_B, _S, _HID = 1, 2048, 768
_H, _NOPE, _ROPE, _VD = 12, 64, 32, 64
_QLR, _KVLR = 384, 256
_HI, _DI, _TOPK = 8, 64, 512
_BASE = 10000.0
_NEG = float(jnp.finfo(jnp.float32).min)
_SB = 256


def _mm(a, b):
    return jax.lax.dot_general(a.astype(jnp.bfloat16), b.astype(jnp.bfloat16),
                               (((1,), (1,)), ((), ())),
                               preferred_element_type=jnp.float32)


def _rope(x, cos, sin):
    x1, x2 = x[:, :16], x[:, 16:]
    rot = jnp.concatenate([-x2, x1], axis=1)
    return x * cos + rot * sin


def _proj_body(x_ref, cos_ref, sin_ref, wqa_ref, qnorm_ref, wqb_ref,
               wkva_ref, kvnorm_ref, wkvb_ref, wiqb_ref, wik_ref,
               iknw_ref, iknb_ref, wiw_ref,
               qn_ref, qp_ref, kn_ref, kp_ref, v_ref, iq_ref, ik_ref, iw_ref):
    x = x_ref[...]
    cos = cos_ref[...]
    sin = sin_ref[...]
    qr = _mm(x, wqa_ref[...])
    ms = jnp.mean(qr * qr, axis=1, keepdims=True)
    qr = qr * jax.lax.rsqrt(ms + 1e-6) * qnorm_ref[...]
    q = _mm(qr, wqb_ref[...])
    for h in range(_H):
        base = h * (_NOPE + _ROPE)
        qn_ref[h] = q[:, base:base + _NOPE]
        qp_ref[h] = _rope(q[:, base + _NOPE:base + _NOPE + _ROPE], cos, sin)
    kva = _mm(x, wkva_ref[...])
    ckv = kva[:, :_KVLR]
    ms = jnp.mean(ckv * ckv, axis=1, keepdims=True)
    ckv = ckv * jax.lax.rsqrt(ms + 1e-6) * kvnorm_ref[...]
    kp_ref[...] = _rope(kva[:, _KVLR:], cos, sin)
    kv = _mm(ckv, wkvb_ref[...])
    for h in range(_H):
        base = h * (_NOPE + _VD)
        kn_ref[h] = kv[:, base:base + _NOPE]
        v_ref[h] = kv[:, base + _NOPE:base + _NOPE + _VD]
    iq = _mm(qr, wiqb_ref[...])
    for h in range(_HI):
        base = h * _DI
        iq_ref[h, :, :_ROPE] = _rope(iq[:, base:base + _ROPE], cos, sin)
        iq_ref[h, :, _ROPE:] = iq[:, base + _ROPE:base + _DI]
    ikx = _mm(x, wik_ref[...])
    m = jnp.mean(ikx, axis=1, keepdims=True)
    var = jnp.mean((ikx - m) ** 2, axis=1, keepdims=True)
    ikx = (ikx - m) * jax.lax.rsqrt(var + 1e-6) * iknw_ref[...] + iknb_ref[...]
    ik_ref[:, :_ROPE] = _rope(ikx[:, :_ROPE], cos, sin)
    ik_ref[:, _ROPE:] = ikx[:, _ROPE:]
    iw_ref[...] = _mm(x, wiw_ref[...]) * (_HI ** -0.5)


def _attn_body(qn_ref, qp_ref, iq_ref, iw_ref, kn_ref, kp_ref, v_ref,
               ik_ref, wo_ref, out_ref, ao_ref):
    i = pl.program_id(0)
    iw = iw_ref[...].astype(jnp.bfloat16).astype(jnp.float32)
    ikv = ik_ref[...]
    acc = jnp.zeros((_SB, _S), jnp.float32)
    for h in range(_HI):
        sh = _mm(iq_ref[h], ikv)
        shb = jnp.maximum(sh, 0.0).astype(jnp.bfloat16).astype(jnp.float32)
        acc = acc + iw[:, h:h + 1] * shb
    qpos = i * _SB + jax.lax.broadcasted_iota(jnp.int32, (_SB, _S), 0)
    kpos = jax.lax.broadcasted_iota(jnp.int32, (_SB, _S), 1)
    causal = qpos >= kpos
    scores = jnp.where(causal, acc, _NEG)
    bits = jax.lax.bitcast_convert_type(scores, jnp.int32)
    bits = jnp.where(bits == jnp.int32(-2147483648), 0, bits)
    ukey = jax.lax.bitcast_convert_type(
        jnp.where(bits >= 0, bits | jnp.int32(-2147483648), ~bits),
        jnp.uint32)
    thr = jnp.zeros((_SB, 1), jnp.uint32)
    for b in range(31, -1, -1):
        cand = thr | jnp.uint32(1 << b)
        cnt = jnp.sum((ukey >= cand).astype(jnp.int32), axis=1, keepdims=True)
        thr = jnp.where(cnt >= _TOPK, cand, thr)
    keep = (ukey >= thr) & causal
    scale = (_NOPE + _ROPE) ** -0.5
    kpv = kp_ref[...]
    for h in range(_H):
        lg = (_mm(qn_ref[h], kn_ref[h]) + _mm(qp_ref[h], kpv)) * scale
        lg = jnp.where(keep, lg, _NEG)
        m = jnp.max(lg, axis=1, keepdims=True)
        e = jnp.exp(lg - m)
        s = jnp.sum(e, axis=1, keepdims=True)
        p = e / s
        o = jax.lax.dot_general(p.astype(jnp.bfloat16),
                                v_ref[h].astype(jnp.bfloat16),
                                (((1,), (0,)), ((), ())),
                                preferred_element_type=jnp.float32)
        ao_ref[:, h * _VD:(h + 1) * _VD] = o
    out_ref[...] = _mm(ao_ref[...], wo_ref[...])


def kernel(hidden_states, position_ids, w_q_a, q_a_norm_w, w_q_b, w_kv_a,
           kv_a_norm_w, w_kv_b, w_o, w_idx_qb, w_idx_k, idx_k_norm_w,
           idx_k_norm_b, w_idx_w):
    x = hidden_states.reshape(_S, _HID)
    inv_freq = 1.0 / (_BASE ** (jnp.arange(0, _ROPE, 2, dtype=jnp.float32) / _ROPE))
    t = position_ids.reshape(_S).astype(jnp.float32)
    freqs = t[:, None] * inv_freq[None, :]
    emb = jnp.concatenate([freqs, freqs], axis=-1)
    cos, sin = jnp.cos(emb), jnp.sin(emb)

    nblk = _S // _SB
    row_spec = lambda d: pl.BlockSpec((_SB, d), lambda i: (i, 0))
    head_spec = lambda nh, d: pl.BlockSpec((nh, _SB, d), lambda i: (0, i, 0))
    full2 = lambda a, b: pl.BlockSpec((a, b), lambda i: (0, 0))
    full3 = lambda a, b, c: pl.BlockSpec((a, b, c), lambda i: (0, 0, 0))

    qn, qp, kn, kp, v, iq, ik, iw = pl.pallas_call(
        _proj_body,
        grid=(nblk,),
        in_specs=[
            row_spec(_HID), row_spec(_ROPE), row_spec(_ROPE),
            full2(_QLR, _HID), full2(1, _QLR), full2(_H * (_NOPE + _ROPE), _QLR),
            full2(_KVLR + _ROPE, _HID), full2(1, _KVLR),
            full2(_H * (_NOPE + _VD), _KVLR),
            full2(_HI * _DI, _QLR), full2(_DI, _HID),
            full2(1, _DI), full2(1, _DI), full2(_HI, _HID),
        ],
        out_specs=[
            head_spec(_H, _NOPE), head_spec(_H, _ROPE), head_spec(_H, _NOPE),
            row_spec(_ROPE), head_spec(_H, _VD), head_spec(_HI, _DI),
            row_spec(_DI), row_spec(_HI),
        ],
        out_shape=[
            jax.ShapeDtypeStruct((_H, _S, _NOPE), jnp.float32),
            jax.ShapeDtypeStruct((_H, _S, _ROPE), jnp.float32),
            jax.ShapeDtypeStruct((_H, _S, _NOPE), jnp.float32),
            jax.ShapeDtypeStruct((_S, _ROPE), jnp.float32),
            jax.ShapeDtypeStruct((_H, _S, _VD), jnp.float32),
            jax.ShapeDtypeStruct((_HI, _S, _DI), jnp.float32),
            jax.ShapeDtypeStruct((_S, _DI), jnp.float32),
            jax.ShapeDtypeStruct((_S, _HI), jnp.float32),
        ],
    )(x, cos, sin, w_q_a, q_a_norm_w.reshape(1, _QLR), w_q_b,
      w_kv_a, kv_a_norm_w.reshape(1, _KVLR), w_kv_b, w_idx_qb, w_idx_k,
      idx_k_norm_w.reshape(1, _DI), idx_k_norm_b.reshape(1, _DI), w_idx_w)

    out = pl.pallas_call(
        _attn_body,
        grid=(nblk,),
        in_specs=[
            head_spec(_H, _NOPE), head_spec(_H, _ROPE), head_spec(_HI, _DI),
            row_spec(_HI),
            full3(_H, _S, _NOPE), full2(_S, _ROPE), full3(_H, _S, _VD),
            full2(_S, _DI), full2(_HID, _H * _VD),
        ],
        out_specs=row_spec(_HID),
        out_shape=jax.ShapeDtypeStruct((_S, _HID), jnp.float32),
        scratch_shapes=[pltpu.VMEM((_SB, _H * _VD), jnp.float32)],
        compiler_params=pltpu.CompilerParams(
            vmem_limit_bytes=100 * 1024 * 1024),
    )(qn, qp, iq, iw, kn, kp, v, ik, w_o)

    return out.reshape(_B, _S, _HID)
